# baseline (speedup 1.0000x reference)
import numpy as np
import jax
import jax.numpy as jnp
from jax.experimental import pallas as pl

N = 32768
N1 = 128
N2 = 256
_TOPK = 8


def _fft_topk_kernel(x_ref, win_ref, f1r_ref, f1i_ref, f2r_ref, f2i_ref,
                     twr_ref, twi_ref, o_ref):
    a = x_ref[0] * win_ref[...]
    hp = jax.lax.Precision.HIGHEST
    f32 = jnp.float32
    br = jax.lax.dot(f1r_ref[...], a, precision=hp, preferred_element_type=f32)
    bi = jax.lax.dot(f1i_ref[...], a, precision=hp, preferred_element_type=f32)
    cr = br * twr_ref[...] - bi * twi_ref[...]
    ci = br * twi_ref[...] + bi * twr_ref[...]
    dr = (jax.lax.dot(cr, f2r_ref[...], precision=hp, preferred_element_type=f32)
          - jax.lax.dot(ci, f2i_ref[...], precision=hp, preferred_element_type=f32))
    di = (jax.lax.dot(cr, f2i_ref[...], precision=hp, preferred_element_type=f32)
          + jax.lax.dot(ci, f2r_ref[...], precision=hp, preferred_element_type=f32))
    mag2 = dr * dr + di * di

    row = jax.lax.broadcasted_iota(jnp.int32, (N1, N2), 0)
    col = jax.lax.broadcasted_iota(jnp.int32, (N1, N2), 1)
    tflat = row * N2 + col

    def body(_, carry):
        m2, acc = carry
        mx = jnp.max(m2)
        p = jnp.min(jnp.where(m2 == mx, tflat, jnp.int32(2 ** 30)))
        freq = (p >> 8) + ((p & 255) << 7)
        val = jnp.sqrt(mx)
        ph = (tflat * freq) & (N - 1)
        acc = acc + val * jnp.cos(ph.astype(f32) * f32(2.0 * np.pi / N))
        m2 = jnp.where(tflat == p, f32(-1.0), m2)
        return m2, acc

    _, acc = jax.lax.fori_loop(
        0, _TOPK, body, (mag2, jnp.zeros((N1, N2), f32)), unroll=True)
    o_ref[0] = acc * f32(1.0 / N)


def _constants():
    n = np.arange(N)
    win = (0.5 * (1.0 - np.cos(2.0 * np.pi * n / N))).astype(np.float32)
    i1 = np.arange(N1)
    i2 = np.arange(N2)
    f1 = np.exp(-2j * np.pi * np.outer(i1, i1) / N1)
    f2 = np.exp(-2j * np.pi * np.outer(i2, i2) / N2)
    tw = np.exp(-2j * np.pi * np.outer(i1, i2) / N)
    return (win.reshape(N1, N2),
            f1.real.astype(np.float32), f1.imag.astype(np.float32),
            f2.real.astype(np.float32), f2.imag.astype(np.float32),
            tw.real.astype(np.float32), tw.imag.astype(np.float32))


def kernel(inputs):
    x = inputs[:, :, 0]
    b = x.shape[0]
    x3 = x.reshape(b, N1, N2)
    consts = _constants()
    out = pl.pallas_call(
        _fft_topk_kernel,
        grid=(b,),
        in_specs=[pl.BlockSpec((1, N1, N2), lambda i: (i, 0, 0))]
                 + [pl.BlockSpec(c.shape, lambda i: (0, 0)) for c in consts],
        out_specs=pl.BlockSpec((1, N1, N2), lambda i: (i, 0, 0)),
        out_shape=jax.ShapeDtypeStruct((b, N1, N2), jnp.float32),
    )(x3, *consts)
    return out.reshape(b, N)[:, :, None]


# speedup vs baseline: 1.2775x; 1.2775x over previous
import numpy as np
import jax
import jax.numpy as jnp
from jax.experimental import pallas as pl

N = 32768
N1 = 128
N2 = 256
_TOPK = 8


def _fft_topk_kernel(x_ref, win_ref, f1r_ref, f1i_ref, f2r_ref, f2i_ref,
                     twr_ref, twi_ref, o_ref):
    a = x_ref[0] * win_ref[...]
    hp = jax.lax.Precision.HIGHEST
    f32 = jnp.float32
    br = jax.lax.dot(f1r_ref[...], a, precision=hp, preferred_element_type=f32)
    bi = jax.lax.dot(f1i_ref[...], a, precision=hp, preferred_element_type=f32)
    cr = br * twr_ref[...] - bi * twi_ref[...]
    ci = br * twi_ref[...] + bi * twr_ref[...]
    dr = (jax.lax.dot(cr, f2r_ref[...], precision=hp, preferred_element_type=f32)
          - jax.lax.dot(ci, f2i_ref[...], precision=hp, preferred_element_type=f32))
    di = (jax.lax.dot(cr, f2i_ref[...], precision=hp, preferred_element_type=f32)
          + jax.lax.dot(ci, f2r_ref[...], precision=hp, preferred_element_type=f32))
    mag2 = dr * dr + di * di

    row = jax.lax.broadcasted_iota(jnp.int32, (N1, N2), 0)
    col = jax.lax.broadcasted_iota(jnp.int32, (N1, N2), 1)
    tflat = row * N2 + col

    m2 = mag2
    freqs, vals = [], []
    for _ in range(_TOPK):
        mx = jnp.max(m2)
        p = jnp.min(jnp.where(m2 == mx, tflat, jnp.int32(2 ** 30)))
        freqs.append((p >> 8) + ((p & 255) << 7))
        vals.append(jnp.sqrt(mx))
        m2 = jnp.where(tflat == p, f32(-1.0), m2)

    crow = jax.lax.broadcasted_iota(jnp.int32, (1, 2 * _TOPK), 1)
    rrow = jax.lax.broadcasted_iota(jnp.int32, (2 * _TOPK, 1), 0)
    kvec = jnp.zeros((1, 2 * _TOPK), jnp.int32)
    vvec = jnp.zeros((1, 2 * _TOPK), f32)
    kcol = jnp.zeros((2 * _TOPK, 1), jnp.int32)
    for j in range(_TOPK):
        kvec = jnp.where((crow >> 1) == j, freqs[j], kvec)
        vvec = jnp.where((crow >> 1) == j, vals[j], vvec)
        kcol = jnp.where((rrow >> 1) == j, freqs[j], kcol)
    rad = f32(2.0 * np.pi / N)
    i1v = jax.lax.broadcasted_iota(jnp.int32, (N1, 1), 0)
    ang_a = (((i1v * N2) * kvec) & (N - 1)).astype(f32) * rad
    u = jnp.where((crow & 1) == 0, vvec * jnp.cos(ang_a),
                  -vvec * jnp.sin(ang_a))
    n2v = jax.lax.broadcasted_iota(jnp.int32, (1, N2), 1)
    ang_b = ((kcol * n2v) & (N - 1)).astype(f32) * rad
    v = jnp.where((rrow & 1) == 0, jnp.cos(ang_b), jnp.sin(ang_b))
    acc = jax.lax.dot(u, v, precision=hp, preferred_element_type=f32)
    o_ref[0] = acc * f32(1.0 / N)


def _constants():
    n = np.arange(N)
    win = (0.5 * (1.0 - np.cos(2.0 * np.pi * n / N))).astype(np.float32)
    i1 = np.arange(N1)
    i2 = np.arange(N2)
    f1 = np.exp(-2j * np.pi * np.outer(i1, i1) / N1)
    f2 = np.exp(-2j * np.pi * np.outer(i2, i2) / N2)
    tw = np.exp(-2j * np.pi * np.outer(i1, i2) / N)
    return (win.reshape(N1, N2),
            f1.real.astype(np.float32), f1.imag.astype(np.float32),
            f2.real.astype(np.float32), f2.imag.astype(np.float32),
            tw.real.astype(np.float32), tw.imag.astype(np.float32))


def kernel(inputs):
    x = inputs[:, :, 0]
    b = x.shape[0]
    x3 = x.reshape(b, N1, N2)
    consts = _constants()
    out = pl.pallas_call(
        _fft_topk_kernel,
        grid=(b,),
        in_specs=[pl.BlockSpec((1, N1, N2), lambda i: (i, 0, 0))]
                 + [pl.BlockSpec(c.shape, lambda i: (0, 0)) for c in consts],
        out_specs=pl.BlockSpec((1, N1, N2), lambda i: (i, 0, 0)),
        out_shape=jax.ShapeDtypeStruct((b, N1, N2), jnp.float32),
    )(x3, *consts)
    return out.reshape(b, N)[:, :, None]


# speedup vs baseline: 1.3296x; 1.0408x over previous
import numpy as np
import jax
import jax.numpy as jnp
from jax.experimental import pallas as pl

N = 32768
N1 = 128
N2 = 256
_TOPK = 8


ROWS = 4


def _fft_topk_kernel(x_ref, win_ref, f1r_ref, f1i_ref, f2r_ref, f2i_ref,
                     twr_ref, twi_ref, o_ref):
    for r in range(ROWS):
        _one_row(x_ref, win_ref, f1r_ref, f1i_ref, f2r_ref, f2i_ref,
                 twr_ref, twi_ref, o_ref, r)


def _one_row(x_ref, win_ref, f1r_ref, f1i_ref, f2r_ref, f2i_ref,
             twr_ref, twi_ref, o_ref, r):
    a = x_ref[r] * win_ref[...]
    hp = jax.lax.Precision.HIGHEST
    f32 = jnp.float32
    br = jax.lax.dot(f1r_ref[...], a, precision=hp, preferred_element_type=f32)
    bi = jax.lax.dot(f1i_ref[...], a, precision=hp, preferred_element_type=f32)
    cr = br * twr_ref[...] - bi * twi_ref[...]
    ci = br * twi_ref[...] + bi * twr_ref[...]
    dr = (jax.lax.dot(cr, f2r_ref[...], precision=hp, preferred_element_type=f32)
          - jax.lax.dot(ci, f2i_ref[...], precision=hp, preferred_element_type=f32))
    di = (jax.lax.dot(cr, f2i_ref[...], precision=hp, preferred_element_type=f32)
          + jax.lax.dot(ci, f2r_ref[...], precision=hp, preferred_element_type=f32))
    mag2 = dr * dr + di * di

    row = jax.lax.broadcasted_iota(jnp.int32, (N1, N2), 0)
    col = jax.lax.broadcasted_iota(jnp.int32, (N1, N2), 1)
    tflat = row * N2 + col

    m2 = mag2
    freqs, vals = [], []
    for _ in range(_TOPK):
        mx = jnp.max(m2)
        p = jnp.min(jnp.where(m2 == mx, tflat, jnp.int32(2 ** 30)))
        freqs.append((p >> 8) + ((p & 255) << 7))
        vals.append(jnp.sqrt(mx))
        m2 = jnp.where(tflat == p, f32(-1.0), m2)

    crow = jax.lax.broadcasted_iota(jnp.int32, (1, 2 * _TOPK), 1)
    rrow = jax.lax.broadcasted_iota(jnp.int32, (2 * _TOPK, 1), 0)
    kvec = jnp.zeros((1, 2 * _TOPK), jnp.int32)
    vvec = jnp.zeros((1, 2 * _TOPK), f32)
    kcol = jnp.zeros((2 * _TOPK, 1), jnp.int32)
    for j in range(_TOPK):
        kvec = jnp.where((crow >> 1) == j, freqs[j], kvec)
        vvec = jnp.where((crow >> 1) == j, vals[j], vvec)
        kcol = jnp.where((rrow >> 1) == j, freqs[j], kcol)
    rad = f32(2.0 * np.pi / N)
    i1v = jax.lax.broadcasted_iota(jnp.int32, (N1, 1), 0)
    ang_a = (((i1v * N2) * kvec) & (N - 1)).astype(f32) * rad
    u = jnp.where((crow & 1) == 0, vvec * jnp.cos(ang_a),
                  -vvec * jnp.sin(ang_a))
    n2v = jax.lax.broadcasted_iota(jnp.int32, (1, N2), 1)
    ang_b = ((kcol * n2v) & (N - 1)).astype(f32) * rad
    v = jnp.where((rrow & 1) == 0, jnp.cos(ang_b), jnp.sin(ang_b))
    acc = jax.lax.dot(u, v, precision=hp, preferred_element_type=f32)
    o_ref[r] = acc * f32(1.0 / N)


def _constants():
    n = np.arange(N)
    win = (0.5 * (1.0 - np.cos(2.0 * np.pi * n / N))).astype(np.float32)
    i1 = np.arange(N1)
    i2 = np.arange(N2)
    f1 = np.exp(-2j * np.pi * np.outer(i1, i1) / N1)
    f2 = np.exp(-2j * np.pi * np.outer(i2, i2) / N2)
    tw = np.exp(-2j * np.pi * np.outer(i1, i2) / N)
    return (win.reshape(N1, N2),
            f1.real.astype(np.float32), f1.imag.astype(np.float32),
            f2.real.astype(np.float32), f2.imag.astype(np.float32),
            tw.real.astype(np.float32), tw.imag.astype(np.float32))


def kernel(inputs):
    x = inputs[:, :, 0]
    b = x.shape[0]
    x3 = x.reshape(b, N1, N2)
    consts = _constants()
    out = pl.pallas_call(
        _fft_topk_kernel,
        grid=(b // ROWS,),
        in_specs=[pl.BlockSpec((ROWS, N1, N2), lambda i: (i, 0, 0))]
                 + [pl.BlockSpec(c.shape, lambda i: (0, 0)) for c in consts],
        out_specs=pl.BlockSpec((ROWS, N1, N2), lambda i: (i, 0, 0)),
        out_shape=jax.ShapeDtypeStruct((b, N1, N2), jnp.float32),
    )(x3, *consts)
    return out.reshape(b, N)[:, :, None]


# speedup vs baseline: 2.5245x; 1.8986x over previous
import numpy as np
import jax
import jax.numpy as jnp
from jax.experimental import pallas as pl

N = 32768
N1 = 128
N2 = 256
_TOPK = 8


ROWS = 4


def _fft_topk_kernel(x_ref, win_ref, f1r_ref, f1i_ref, f2r_ref, f2i_ref,
                     twr_ref, twi_ref, o_ref):
    hp = jax.lax.Precision.HIGHEST
    f32 = jnp.float32
    R = range(ROWS)

    def dot(a, b):
        return jax.lax.dot(a, b, precision=hp, preferred_element_type=f32)

    win = win_ref[...]
    f1r, f1i = f1r_ref[...], f1i_ref[...]
    f2r, f2i = f2r_ref[...], f2i_ref[...]
    twr, twi = twr_ref[...], twi_ref[...]

    a = [x_ref[r] * win for r in R]
    br = [dot(f1r, a[r]) for r in R]
    bi = [dot(f1i, a[r]) for r in R]
    cr = [br[r] * twr - bi[r] * twi for r in R]
    ci = [br[r] * twi + bi[r] * twr for r in R]
    dr = [dot(cr[r], f2r) - dot(ci[r], f2i) for r in R]
    di = [dot(cr[r], f2i) + dot(ci[r], f2r) for r in R]
    m2 = [dr[r] * dr[r] + di[r] * di[r] for r in R]

    row = jax.lax.broadcasted_iota(jnp.int32, (N1, N2), 0)
    col = jax.lax.broadcasted_iota(jnp.int32, (N1, N2), 1)
    tflat = row * N2 + col

    freqs = [[] for _ in R]
    vals = [[] for _ in R]
    for _ in range(_TOPK):
        mx = [jnp.max(m2[r]) for r in R]
        p = [jnp.min(jnp.where(m2[r] == mx[r], tflat, jnp.int32(2 ** 30)))
             for r in R]
        for r in R:
            freqs[r].append((p[r] >> 8) + ((p[r] & 255) << 7))
            vals[r].append(jnp.sqrt(mx[r]))
        m2 = [jnp.where(tflat == p[r], f32(-1.0), m2[r]) for r in R]

    crow = jax.lax.broadcasted_iota(jnp.int32, (1, 2 * _TOPK), 1)
    rrow = jax.lax.broadcasted_iota(jnp.int32, (2 * _TOPK, 1), 0)
    kvec = [jnp.zeros((1, 2 * _TOPK), jnp.int32) for _ in R]
    vvec = [jnp.zeros((1, 2 * _TOPK), f32) for _ in R]
    kcol = [jnp.zeros((2 * _TOPK, 1), jnp.int32) for _ in R]
    for j in range(_TOPK):
        csel = (crow >> 1) == j
        rsel = (rrow >> 1) == j
        for r in R:
            kvec[r] = jnp.where(csel, freqs[r][j], kvec[r])
            vvec[r] = jnp.where(csel, vals[r][j], vvec[r])
            kcol[r] = jnp.where(rsel, freqs[r][j], kcol[r])
    rad = f32(2.0 * np.pi / N)
    i1v = jax.lax.broadcasted_iota(jnp.int32, (N1, 1), 0)
    n2v = jax.lax.broadcasted_iota(jnp.int32, (1, N2), 1)
    ceven = (crow & 1) == 0
    reven = (rrow & 1) == 0
    ang_a = [(((i1v * N2) * kvec[r]) & (N - 1)).astype(f32) * rad for r in R]
    u = [jnp.where(ceven, vvec[r] * jnp.cos(ang_a[r]),
                   -vvec[r] * jnp.sin(ang_a[r])) for r in R]
    ang_b = [((kcol[r] * n2v) & (N - 1)).astype(f32) * rad for r in R]
    v = [jnp.where(reven, jnp.cos(ang_b[r]), jnp.sin(ang_b[r])) for r in R]
    for r in R:
        o_ref[r] = dot(u[r], v[r]) * f32(1.0 / N)


def _constants():
    n = np.arange(N)
    win = (0.5 * (1.0 - np.cos(2.0 * np.pi * n / N))).astype(np.float32)
    i1 = np.arange(N1)
    i2 = np.arange(N2)
    f1 = np.exp(-2j * np.pi * np.outer(i1, i1) / N1)
    f2 = np.exp(-2j * np.pi * np.outer(i2, i2) / N2)
    tw = np.exp(-2j * np.pi * np.outer(i1, i2) / N)
    return (win.reshape(N1, N2),
            f1.real.astype(np.float32), f1.imag.astype(np.float32),
            f2.real.astype(np.float32), f2.imag.astype(np.float32),
            tw.real.astype(np.float32), tw.imag.astype(np.float32))


def kernel(inputs):
    x = inputs[:, :, 0]
    b = x.shape[0]
    x3 = x.reshape(b, N1, N2)
    consts = _constants()
    out = pl.pallas_call(
        _fft_topk_kernel,
        grid=(b // ROWS,),
        in_specs=[pl.BlockSpec((ROWS, N1, N2), lambda i: (i, 0, 0))]
                 + [pl.BlockSpec(c.shape, lambda i: (0, 0)) for c in consts],
        out_specs=pl.BlockSpec((ROWS, N1, N2), lambda i: (i, 0, 0)),
        out_shape=jax.ShapeDtypeStruct((b, N1, N2), jnp.float32),
    )(x3, *consts)
    return out.reshape(b, N)[:, :, None]
